# speedup vs baseline: 10.9737x; 10.9737x over previous
import jax
import jax.numpy as jnp
from jax import lax
from jax.experimental import pallas as pl
from jax.experimental.pallas import tpu as pltpu
from jax.experimental.pallas import tpu_sc as plsc

N = 10000
D = 128
H = 128
A = 64
B = 1024
E = 320000

NC, NS = 2, 16
NW = NC * NS
N_PAD = 10240
CH = 128
E_PAD = -(-E // (NW * CH)) * (NW * CH)
EW = E_PAD // NW
CHUNKS = EW // CH
RPT = N_PAD // NS
DEGW = 8

_mesh = plsc.VectorSubcoreMesh(core_axis_name="c", subcore_axis_name="s")


def _deg_body(dst_hbm, zeros_hbm, ones_hbm, out_hbm, idx_v, ones_v, acc):
    c = lax.axis_index("c")
    s = lax.axis_index("s")
    wid = s * NC + c
    pltpu.sync_copy(zeros_hbm.at[pl.ds(s * RPT, RPT)], acc.at[pl.ds(s * RPT, RPT)])
    pltpu.sync_copy(ones_hbm, ones_v)
    plsc.subcore_barrier()
    base = wid * EW

    def body(i, carry):
        pltpu.sync_copy(dst_hbm.at[pl.ds(base + i * CH, CH)], idx_v)
        pltpu.sync_copy(ones_v, acc.at[idx_v], add=True)
        return carry

    lax.fori_loop(0, CHUNKS, body, 0)
    plsc.subcore_barrier()
    pltpu.sync_copy(acc.at[pl.ds(s * RPT, RPT)], out_hbm.at[c, pl.ds(s * RPT, RPT)])


_deg_call = pl.kernel(
    _deg_body,
    out_type=jax.ShapeDtypeStruct((NC, N_PAD, DEGW), jnp.float32),
    mesh=_mesh,
    scratch_types=[
        pltpu.VMEM((CH,), jnp.int32),
        pltpu.VMEM((CH, DEGW), jnp.float32),
        pltpu.VMEM_SHARED((N_PAD, DEGW), jnp.float32),
    ],
)


def _mp_body(g_hbm, src_hbm, dst_hbm, zeros_hbm, out_hbm, srcv, dstv, rows, acc, sem):
    c = lax.axis_index("c")
    s = lax.axis_index("s")
    wid = s * NC + c
    pltpu.sync_copy(zeros_hbm.at[pl.ds(s * RPT, RPT)], acc.at[pl.ds(s * RPT, RPT)])
    plsc.subcore_barrier()
    base = wid * EW

    def body(i, carry):
        off = base + i * CH
        pltpu.sync_copy(src_hbm.at[pl.ds(off, CH)], srcv)
        pltpu.sync_copy(dst_hbm.at[pl.ds(off, CH)], dstv)
        pltpu.async_copy(g_hbm.at[srcv], rows, sem).wait()
        pltpu.sync_copy(rows, acc.at[dstv], add=True)
        return carry

    lax.fori_loop(0, CHUNKS, body, 0)
    plsc.subcore_barrier()
    pltpu.sync_copy(acc.at[pl.ds(s * RPT, RPT)], out_hbm.at[c, pl.ds(s * RPT, RPT)])


_mp_call = pl.kernel(
    _mp_body,
    out_type=jax.ShapeDtypeStruct((NC, N_PAD, H), jnp.float32),
    mesh=_mesh,
    scratch_types=[
        pltpu.VMEM((CH,), jnp.int32),
        pltpu.VMEM((CH,), jnp.int32),
        pltpu.VMEM((CH, H), jnp.float32),
        pltpu.VMEM_SHARED((N_PAD, H), jnp.float32),
        pltpu.SemaphoreType.DMA,
    ],
)


BN = 1024
GN = N_PAD // BN


def _dinv_of(deg_blk):
    d = deg_blk[0] + deg_blk[1]
    return lax.rsqrt(d[:, 0:1] + 1.0)


def _l1_body(deg_ref, x_ref, w1_ref, g1_ref):
    dinv = _dinv_of(deg_ref)
    g = lax.dot_general(x_ref[...], w1_ref[...], (((0,), (0,)), ((), ())),
                        preferred_element_type=jnp.float32)
    g1_ref[...] = g * dinv


_l1_call = pl.pallas_call(
    _l1_body,
    grid=(GN,),
    in_specs=[
        pl.BlockSpec((NC, BN, DEGW), lambda i: (0, i, 0)),
        pl.BlockSpec((D, BN), lambda i: (0, i)),
        pl.BlockSpec((D, H), lambda i: (0, 0)),
    ],
    out_specs=pl.BlockSpec((BN, H), lambda i: (i, 0)),
    out_shape=jax.ShapeDtypeStruct((N_PAD, H), jnp.float32),
)


def _l2_body(deg_ref, s1_ref, g1_ref, w2_ref, b1_ref, g2_ref):
    dinv = _dinv_of(deg_ref)
    h1 = jnp.maximum(dinv * (s1_ref[0] + s1_ref[1] + g1_ref[...]) + b1_ref[...], 0.0)
    g2_ref[...] = dinv * jnp.dot(h1, w2_ref[...], preferred_element_type=jnp.float32)


_l2_call = pl.pallas_call(
    _l2_body,
    grid=(GN,),
    in_specs=[
        pl.BlockSpec((NC, BN, DEGW), lambda i: (0, i, 0)),
        pl.BlockSpec((NC, BN, H), lambda i: (0, i, 0)),
        pl.BlockSpec((BN, H), lambda i: (i, 0)),
        pl.BlockSpec((H, H), lambda i: (0, 0)),
        pl.BlockSpec((1, H), lambda i: (0, 0)),
    ],
    out_specs=pl.BlockSpec((BN, H), lambda i: (i, 0)),
    out_shape=jax.ShapeDtypeStruct((N_PAD, H), jnp.float32),
)


def _l3_body(deg_ref, s2_ref, g2_ref, b2_ref, h2_ref):
    dinv = _dinv_of(deg_ref)
    h2_ref[...] = dinv * (s2_ref[0] + s2_ref[1] + g2_ref[...]) + b2_ref[...]


_l3_call = pl.pallas_call(
    _l3_body,
    grid=(GN,),
    in_specs=[
        pl.BlockSpec((NC, BN, DEGW), lambda i: (0, i, 0)),
        pl.BlockSpec((NC, BN, H), lambda i: (0, i, 0)),
        pl.BlockSpec((BN, H), lambda i: (i, 0)),
        pl.BlockSpec((1, H), lambda i: (0, 0)),
    ],
    out_specs=pl.BlockSpec((BN, H), lambda i: (i, 0)),
    out_shape=jax.ShapeDtypeStruct((N_PAD, H), jnp.float32),
)


BB = 256


def _out_body(o_ref, h2_ref, out_ref):
    out_ref[...] = jnp.dot(o_ref[...], h2_ref[...], preferred_element_type=jnp.float32)


_out_call = pl.pallas_call(
    _out_body,
    grid=(B // BB,),
    in_specs=[
        pl.BlockSpec((BB, N), lambda i: (i, 0)),
        pl.BlockSpec((N, A), lambda i: (0, 0)),
    ],
    out_specs=pl.BlockSpec((BB, A), lambda i: (i, 0)),
    out_shape=jax.ShapeDtypeStruct((B, A), jnp.float32),
)


def kernel(x, edge_index, onehot_values, W1, b1, W2, b2):
    ei = edge_index.astype(jnp.int32)
    pad = jnp.full((E_PAD - E,), N, dtype=jnp.int32)
    src = jnp.concatenate([ei[0], pad])
    dst = jnp.concatenate([ei[1], pad])
    xp = jnp.pad(x, ((0, 0), (0, N_PAD - N)))
    w2p = jnp.pad(W2, ((0, 0), (0, H - A)))
    b1r = b1.reshape(1, H)
    b2r = jnp.pad(b2, (0, H - A)).reshape(1, H)
    zeros_d = jnp.zeros((N_PAD, DEGW), jnp.float32)
    ones_d = jnp.ones((CH, DEGW), jnp.float32)
    zeros_c = jnp.zeros((N_PAD, H), jnp.float32)

    deg = _deg_call(dst, zeros_d, ones_d)
    g1 = _l1_call(deg, xp, W1)
    s1 = _mp_call(g1, src, dst, zeros_c)
    g2 = _l2_call(deg, s1, g1, w2p, b1r)
    s2 = _mp_call(g2, src, dst, zeros_c)
    h2 = _l3_call(deg, s2, g2, b2r)
    return _out_call(onehot_values, h2[:N, :A])
